# baseline (speedup 1.0000x reference)
import functools

import jax
import jax.numpy as jnp
from jax import lax
from jax.experimental import pallas as pl
from jax.experimental.pallas import tpu as pltpu
from jax.experimental.pallas import tpu_sc as plsc

SEQ_LEN = 4096
EMBED_DIM = 1024
BATCH = 4
TOTAL = BATCH * SEQ_LEN

NUM_CORES = 2
NUM_SUBCORES = 16
NUM_WORKERS = NUM_CORES * NUM_SUBCORES

ROWS_PER_WORKER = TOTAL // NUM_WORKERS
CHUNK = 32
N_CHUNKS = ROWS_PER_WORKER // CHUNK

_mesh = plsc.VectorSubcoreMesh(core_axis_name="c", subcore_axis_name="s")


@functools.partial(
    pl.kernel,
    mesh=_mesh,
    out_type=jax.ShapeDtypeStruct((TOTAL, EMBED_DIM), jnp.float32),
    scratch_types=[
        pltpu.VMEM((N_CHUNKS, CHUNK), jnp.int32),
        pltpu.VMEM((3, CHUNK, EMBED_DIM), jnp.float32),
        pltpu.SemaphoreType.DMA,
        pltpu.SemaphoreType.DMA,
    ],
)
def _gather_kernel(table_hbm, idx_hbm, out_hbm, idx_v, bufs, gsem, ssem):
    NBUF = 3
    wid = lax.axis_index("s") * NUM_CORES + lax.axis_index("c")
    base = wid * ROWS_PER_WORKER
    pltpu.sync_copy(idx_hbm.at[wid], idx_v)
    gathers = [None] * NBUF
    scatters = [None] * NBUF
    for j in range(NBUF - 1):
        gathers[j] = pltpu.async_copy(
            table_hbm.at[idx_v.at[j]], bufs.at[j], gsem)
    for j in range(N_CHUNKS):
        b = j % NBUF
        nb = (j + NBUF - 1) % NBUF
        if j + NBUF - 1 < N_CHUNKS:
            if scatters[nb] is not None:
                scatters[nb].wait()
            gathers[nb] = pltpu.async_copy(
                table_hbm.at[idx_v.at[j + NBUF - 1]], bufs.at[nb], gsem)
        gathers[b].wait()
        scatters[b] = pltpu.async_copy(
            bufs.at[b], out_hbm.at[pl.ds(base + j * CHUNK, CHUNK)], ssem)
    for j in range(NBUF):
        scatters[(N_CHUNKS - NBUF + j) % NBUF].wait()


def kernel(input_positions, position_embeddings):
    idx = jnp.reshape(input_positions.astype(jnp.int32),
                      (NUM_WORKERS, N_CHUNKS, CHUNK))
    out = _gather_kernel(position_embeddings, idx)
    return jnp.reshape(out, (BATCH, SEQ_LEN, EMBED_DIM))


# speedup vs baseline: 1.0042x; 1.0042x over previous
import functools

import jax
import jax.numpy as jnp
from jax import lax
from jax.experimental import pallas as pl
from jax.experimental.pallas import tpu as pltpu
from jax.experimental.pallas import tpu_sc as plsc

SEQ_LEN = 4096
EMBED_DIM = 1024
BATCH = 4
TOTAL = BATCH * SEQ_LEN

NUM_CORES = 2
NUM_SUBCORES = 16
NUM_WORKERS = NUM_CORES * NUM_SUBCORES

ROWS_PER_WORKER = TOTAL // NUM_WORKERS
WORKERS_PER_BATCH = SEQ_LEN // ROWS_PER_WORKER
CHUNK = 32
N_CHUNKS = ROWS_PER_WORKER // CHUNK
NBUF = 2

_mesh = plsc.VectorSubcoreMesh(core_axis_name="c", subcore_axis_name="s")


@functools.partial(
    pl.kernel,
    mesh=_mesh,
    out_type=jax.ShapeDtypeStruct((TOTAL, EMBED_DIM), jnp.float32),
    scratch_types=[
        pltpu.VMEM((ROWS_PER_WORKER,), jnp.int32),
        pltpu.VMEM((NBUF, CHUNK, EMBED_DIM), jnp.float32),
        pltpu.SemaphoreType.DMA,
        pltpu.SemaphoreType.DMA,
    ],
)
def _gather_kernel(table_hbm, idx_hbm, out_hbm, idx_v, bufs, gsem, ssem):
    wid = lax.axis_index("s") * NUM_CORES + lax.axis_index("c")
    base = wid * ROWS_PER_WORKER
    b = wid // WORKERS_PER_BATCH
    col = (wid % WORKERS_PER_BATCH) * ROWS_PER_WORKER
    pltpu.sync_copy(idx_hbm.at[b, pl.ds(col, ROWS_PER_WORKER)], idx_v)
    gathers = [None] * NBUF
    scatters = [None] * NBUF
    for j in range(NBUF - 1):
        gathers[j] = pltpu.async_copy(
            table_hbm.at[idx_v.at[pl.ds(j * CHUNK, CHUNK)]], bufs.at[j], gsem)
    for j in range(N_CHUNKS):
        cur = j % NBUF
        nxt = (j + NBUF - 1) % NBUF
        if j + NBUF - 1 < N_CHUNKS:
            if scatters[nxt] is not None:
                scatters[nxt].wait()
            gathers[nxt] = pltpu.async_copy(
                table_hbm.at[idx_v.at[pl.ds((j + NBUF - 1) * CHUNK, CHUNK)]],
                bufs.at[nxt], gsem)
        gathers[cur].wait()
        scatters[cur] = pltpu.async_copy(
            bufs.at[cur], out_hbm.at[pl.ds(base + j * CHUNK, CHUNK)], ssem)
    for j in range(NBUF):
        scatters[(N_CHUNKS - NBUF + j) % NBUF].wait()


def kernel(input_positions, position_embeddings):
    out = _gather_kernel(position_embeddings,
                         input_positions.astype(jnp.int32))
    return jnp.reshape(out, (BATCH, SEQ_LEN, EMBED_DIM))
